# Initial kernel scaffold; baseline (speedup 1.0000x reference)
import jax
import jax.numpy as jnp
from jax.experimental import pallas as pl


def kernel(x, hyperedge_index, W, b):
    raise NotImplementedError("write your pallas kernel here")


# speedup vs baseline: 9.1620x; 9.1620x over previous
import functools

import jax
import jax.numpy as jnp
from jax import lax
from jax.experimental import pallas as pl
from jax.experimental.pallas import tpu as pltpu
from jax.experimental.pallas import tpu_sc as plsc

N = 10000
D = 128
E = 320000
NC, NS, L = 2, 16, 16
NW = NC * NS
EPW = E // NW
CHUNK = 80
ITERS = EPW // CHUNK
RPT = 640
ZR = 40
NZ_FULL = RPT // ZR
NZ_LAST = (N - (NS - 1) * RPT) // ZR


def _mesh():
    return plsc.VectorSubcoreMesh(core_axis_name="c", subcore_axis_name="s",
                                  num_cores=NC, num_subcores=NS)


@functools.lru_cache(maxsize=None)
def _get_hist():
    out_type = [jax.ShapeDtypeStruct((NC, N, D), jnp.float32),
                jax.ShapeDtypeStruct((NC, N, D), jnp.float32)]
    scratch = [
        pltpu.VMEM((1, CHUNK), jnp.int32),
        pltpu.VMEM((CHUNK, D), jnp.float32),
        pltpu.VMEM((ZR, D), jnp.float32),
        pltpu.VMEM_SHARED((N, D), jnp.float32),
    ]

    def body(gidx_h, sidx_h, cntv_h, cnte_h, idx_v, ones_v, zero_v, cw_sh):
        c = lax.axis_index("c")
        s = lax.axis_index("s")
        wid = c * NS + s
        row0 = s * RPT
        nz = jnp.where(s == NS - 1, NZ_LAST, NZ_FULL)

        def fill(i, _):
            for j in range(D // L):
                ones_v[i, pl.ds(j * L, L)] = jnp.ones((L,), jnp.float32)
            return _
        lax.fori_loop(0, CHUNK, fill, 0)

        def fill_zero(i, _):
            for j in range(D // L):
                zero_v[i, pl.ds(j * L, L)] = jnp.zeros((L,), jnp.float32)
            return _
        lax.fori_loop(0, ZR, fill_zero, 0)

        base0 = wid * EPW

        def one_pass(src_h, dst_h):
            def zcopy(i, _):
                pltpu.sync_copy(zero_v, cw_sh.at[pl.ds(row0 + i * ZR, ZR)])
                return _
            lax.fori_loop(0, nz, zcopy, 0)
            plsc.subcore_barrier()

            def step(i, _):
                pltpu.sync_copy(src_h.at[pl.ds(base0 + i * CHUNK, CHUNK)],
                                idx_v.at[0])
                pltpu.sync_copy(ones_v, cw_sh.at[idx_v.at[0]], add=True)
                return _
            lax.fori_loop(0, ITERS, step, 0)
            plsc.subcore_barrier()

            def out_copy(i, _):
                r = row0 + i * ZR
                pltpu.sync_copy(cw_sh.at[pl.ds(r, ZR)], dst_h.at[c, pl.ds(r, ZR)])
                return _
            lax.fori_loop(0, nz, out_copy, 0)
            plsc.subcore_barrier()

        one_pass(gidx_h, cntv_h)
        one_pass(sidx_h, cnte_h)

    return pl.kernel(body, out_type=out_type, mesh=_mesh(),
                     scratch_types=scratch)


@functools.lru_cache(maxsize=None)
def _get_stage():
    out_type = [jax.ShapeDtypeStruct((NC, N, D), jnp.float32)]
    scratch = [
        pltpu.VMEM((1, CHUNK), jnp.int32),
        pltpu.VMEM((1, CHUNK), jnp.int32),
        pltpu.VMEM((CHUNK, D), jnp.float32),
        pltpu.VMEM((ZR, D), jnp.float32),
        pltpu.VMEM_SHARED((N, D), jnp.float32),
        pltpu.SemaphoreType.DMA,
    ]

    def body(table_h, gidx_h, sidx_h, part_h,
             gidx_v, sidx_v, rows_v, zero_v, acc_sh, sem):
        c = lax.axis_index("c")
        s = lax.axis_index("s")
        wid = c * NS + s
        row0 = s * RPT
        nz = jnp.where(s == NS - 1, NZ_LAST, NZ_FULL)

        def fill_zero(i, _):
            for j in range(D // L):
                zero_v[i, pl.ds(j * L, L)] = jnp.zeros((L,), jnp.float32)
            return _
        lax.fori_loop(0, ZR, fill_zero, 0)

        def zcopy(i, _):
            pltpu.sync_copy(zero_v, acc_sh.at[pl.ds(row0 + i * ZR, ZR)])
            return _
        lax.fori_loop(0, nz, zcopy, 0)
        plsc.subcore_barrier()

        base0 = wid * EPW

        def step(i, _):
            base = base0 + i * CHUNK
            pltpu.sync_copy(gidx_h.at[pl.ds(base, CHUNK)], gidx_v.at[0])
            pltpu.sync_copy(sidx_h.at[pl.ds(base, CHUNK)], sidx_v.at[0])
            pltpu.async_copy(table_h.at[gidx_v.at[0]], rows_v, sem).wait()
            pltpu.sync_copy(rows_v, acc_sh.at[sidx_v.at[0]], add=True)
            return _
        lax.fori_loop(0, ITERS, step, 0)
        plsc.subcore_barrier()

        def out_copy(i, _):
            r = row0 + i * ZR
            pltpu.sync_copy(acc_sh.at[pl.ds(r, ZR)], part_h.at[c, pl.ds(r, ZR)])
            return _
        lax.fori_loop(0, nz, out_copy, 0)

    return pl.kernel(body, out_type=out_type, mesh=_mesh(),
                     scratch_types=scratch)


_BLK = 1000


def _recip(c_ref):
    cnt = c_ref[0, :, 0:1] + c_ref[1, :, 0:1]
    return jnp.where(cnt > 0, 1.0 / jnp.where(cnt > 0, cnt, 1.0), 0.0)


def _combine(parts, cnts):
    def body(p_ref, c_ref, o_ref):
        o_ref[...] = (p_ref[0] + p_ref[1]) * _recip(c_ref)

    return pl.pallas_call(
        body,
        grid=(N // _BLK,),
        in_specs=[pl.BlockSpec((NC, _BLK, D), lambda i: (0, i, 0)),
                  pl.BlockSpec((NC, _BLK, D), lambda i: (0, i, 0))],
        out_specs=pl.BlockSpec((_BLK, D), lambda i: (i, 0)),
        out_shape=jax.ShapeDtypeStruct((N, D), jnp.float32),
    )(parts, cnts)


def _final(parts, cnts, W, b2):
    def body(p_ref, c_ref, w_ref, b_ref, o_ref):
        sfeat = (p_ref[0] + p_ref[1]) * _recip(c_ref)
        o_ref[...] = jnp.dot(sfeat, w_ref[...],
                             preferred_element_type=jnp.float32) + b_ref[...]

    return pl.pallas_call(
        body,
        grid=(N // _BLK,),
        in_specs=[pl.BlockSpec((NC, _BLK, D), lambda i: (0, i, 0)),
                  pl.BlockSpec((NC, _BLK, D), lambda i: (0, i, 0)),
                  pl.BlockSpec((D, D), lambda i: (0, 0)),
                  pl.BlockSpec((1, D), lambda i: (0, 0))],
        out_specs=pl.BlockSpec((_BLK, D), lambda i: (i, 0)),
        out_shape=jax.ShapeDtypeStruct((N, D), jnp.float32),
    )(parts, cnts, W, b2)


def kernel(x, hyperedge_index, W, b):
    idx0 = hyperedge_index[0]
    idx1 = hyperedge_index[1]
    cntv, cnte = _get_hist()(idx0, idx1)
    (p1,) = _get_stage()(x, idx0, idx1)
    edge_feat = _combine(p1, cnte)
    (p2,) = _get_stage()(edge_feat, idx1, idx0)
    return _final(p2, cntv, W, b.reshape(1, D))


# speedup vs baseline: 10.1838x; 1.1115x over previous
import functools

import jax
import jax.numpy as jnp
from jax import lax
from jax.experimental import pallas as pl
from jax.experimental.pallas import tpu as pltpu
from jax.experimental.pallas import tpu_sc as plsc

N = 10000
D = 128
E = 320000
NC, NS, L = 2, 16, 16
NW = NC * NS
EPW = E // NW
CHUNK = 80
ITERS = EPW // CHUNK
RPT = 640
ZR = 40
NZ_FULL = RPT // ZR
NZ_LAST = (N - (NS - 1) * RPT) // ZR


def _mesh():
    return plsc.VectorSubcoreMesh(core_axis_name="c", subcore_axis_name="s",
                                  num_cores=NC, num_subcores=NS)


@functools.lru_cache(maxsize=None)
def _get_hist():
    out_type = [jax.ShapeDtypeStruct((NC, N, D), jnp.float32),
                jax.ShapeDtypeStruct((NC, N, D), jnp.float32)]
    scratch = [
        pltpu.VMEM((1, CHUNK), jnp.int32),
        pltpu.VMEM((CHUNK, D), jnp.float32),
        pltpu.VMEM((ZR, D), jnp.float32),
        pltpu.VMEM_SHARED((N, D), jnp.float32),
    ]

    def body(gidx_h, sidx_h, cntv_h, cnte_h, idx_v, ones_v, zero_v, cw_sh):
        c = lax.axis_index("c")
        s = lax.axis_index("s")
        wid = c * NS + s
        row0 = s * RPT
        nz = jnp.where(s == NS - 1, NZ_LAST, NZ_FULL)

        def fill(i, _):
            for j in range(D // L):
                ones_v[i, pl.ds(j * L, L)] = jnp.ones((L,), jnp.float32)
            return _
        lax.fori_loop(0, CHUNK, fill, 0)

        def fill_zero(i, _):
            for j in range(D // L):
                zero_v[i, pl.ds(j * L, L)] = jnp.zeros((L,), jnp.float32)
            return _
        lax.fori_loop(0, ZR, fill_zero, 0)

        base0 = wid * EPW

        def one_pass(src_h, dst_h):
            def zcopy(i, _):
                pltpu.sync_copy(zero_v, cw_sh.at[pl.ds(row0 + i * ZR, ZR)])
                return _
            lax.fori_loop(0, nz, zcopy, 0)
            plsc.subcore_barrier()

            def step(i, _):
                pltpu.sync_copy(src_h.at[pl.ds(base0 + i * CHUNK, CHUNK)],
                                idx_v.at[0])
                pltpu.sync_copy(ones_v, cw_sh.at[idx_v.at[0]], add=True)
                return _
            lax.fori_loop(0, ITERS, step, 0)
            plsc.subcore_barrier()

            def out_copy(i, _):
                r = row0 + i * ZR
                pltpu.sync_copy(cw_sh.at[pl.ds(r, ZR)], dst_h.at[c, pl.ds(r, ZR)])
                return _
            lax.fori_loop(0, nz, out_copy, 0)
            plsc.subcore_barrier()

        one_pass(gidx_h, cntv_h)
        one_pass(sidx_h, cnte_h)

    return pl.kernel(body, out_type=out_type, mesh=_mesh(),
                     scratch_types=scratch)


CK = 128
NCK = 79
EPP = NCK * CK
NDUMP = N + 8


@functools.lru_cache(maxsize=None)
def _get_stage():
    out_type = [jax.ShapeDtypeStruct((NC, N, D), jnp.float32)]
    scratch = [
        pltpu.VMEM((2, 2, CK), jnp.int32),
        pltpu.VMEM((2, CK, D), jnp.float32),
        pltpu.VMEM((ZR, D), jnp.float32),
        pltpu.VMEM_SHARED((NDUMP, D), jnp.float32),
        pltpu.SemaphoreType.DMA,
        pltpu.SemaphoreType.DMA,
    ]

    def body(table_h, pidx_h, part_h, idx_v, rows_v, zero_v, acc_sh,
             sem0, sem1):
        c = lax.axis_index("c")
        s = lax.axis_index("s")
        wid = c * NS + s
        row0 = s * RPT
        nz = jnp.where(s == NS - 1, NZ_LAST, NZ_FULL)
        sems = (sem0, sem1)

        def fill_zero(i, _):
            for j in range(D // L):
                zero_v[i, pl.ds(j * L, L)] = jnp.zeros((L,), jnp.float32)
            return _
        lax.fori_loop(0, ZR, fill_zero, 0)

        def zcopy(i, _):
            pltpu.sync_copy(zero_v, acc_sh.at[pl.ds(row0 + i * ZR, ZR)])
            return _
        lax.fori_loop(0, nz, zcopy, 0)
        plsc.subcore_barrier()

        chunk0 = wid * NCK

        def load_and_fire(i, buf):
            pltpu.sync_copy(pidx_h.at[chunk0 + i], idx_v.at[buf])
            return pltpu.async_copy(table_h.at[idx_v.at[buf, 0]],
                                    rows_v.at[buf], sems[buf])

        def drain(i, buf, cp):
            cp.wait()
            pltpu.sync_copy(rows_v.at[buf], acc_sh.at[idx_v.at[buf, 1]],
                            add=True)

        cp = load_and_fire(0, 0)
        for i in range(NCK):
            buf = i % 2
            nxt = load_and_fire(i + 1, 1 - buf) if i + 1 < NCK else None
            drain(i, buf, cp)
            cp = nxt
        plsc.subcore_barrier()

        def out_copy(i, _):
            r = row0 + i * ZR
            pltpu.sync_copy(acc_sh.at[pl.ds(r, ZR)], part_h.at[c, pl.ds(r, ZR)])
            return _
        lax.fori_loop(0, nz, out_copy, 0)

    return pl.kernel(body, out_type=out_type, mesh=_mesh(),
                     scratch_types=scratch)


def _pack_idx(g, s):
    gw = g.reshape(NW, EPW)
    sw = s.reshape(NW, EPW)
    pad_g = jnp.zeros((NW, EPP - EPW), jnp.int32)
    pad_s = jnp.full((NW, EPP - EPW), N, jnp.int32)
    gp = jnp.concatenate([gw, pad_g], axis=1).reshape(NW, NCK, 1, CK)
    sp = jnp.concatenate([sw, pad_s], axis=1).reshape(NW, NCK, 1, CK)
    return jnp.concatenate([gp, sp], axis=2).reshape(NW * NCK, 2, CK)


_BLK = 1000


def _recip(c_ref):
    cnt = c_ref[0, :, 0:1] + c_ref[1, :, 0:1]
    return jnp.where(cnt > 0, 1.0 / jnp.where(cnt > 0, cnt, 1.0), 0.0)


def _combine(parts, cnts):
    def body(p_ref, c_ref, o_ref):
        o_ref[...] = (p_ref[0] + p_ref[1]) * _recip(c_ref)

    return pl.pallas_call(
        body,
        grid=(N // _BLK,),
        in_specs=[pl.BlockSpec((NC, _BLK, D), lambda i: (0, i, 0)),
                  pl.BlockSpec((NC, _BLK, D), lambda i: (0, i, 0))],
        out_specs=pl.BlockSpec((_BLK, D), lambda i: (i, 0)),
        out_shape=jax.ShapeDtypeStruct((N, D), jnp.float32),
    )(parts, cnts)


def _final(parts, cnts, W, b2):
    def body(p_ref, c_ref, w_ref, b_ref, o_ref):
        sfeat = (p_ref[0] + p_ref[1]) * _recip(c_ref)
        o_ref[...] = jnp.dot(sfeat, w_ref[...],
                             preferred_element_type=jnp.float32) + b_ref[...]

    return pl.pallas_call(
        body,
        grid=(N // _BLK,),
        in_specs=[pl.BlockSpec((NC, _BLK, D), lambda i: (0, i, 0)),
                  pl.BlockSpec((NC, _BLK, D), lambda i: (0, i, 0)),
                  pl.BlockSpec((D, D), lambda i: (0, 0)),
                  pl.BlockSpec((1, D), lambda i: (0, 0))],
        out_specs=pl.BlockSpec((_BLK, D), lambda i: (i, 0)),
        out_shape=jax.ShapeDtypeStruct((N, D), jnp.float32),
    )(parts, cnts, W, b2)


def kernel(x, hyperedge_index, W, b):
    idx0 = hyperedge_index[0]
    idx1 = hyperedge_index[1]
    cntv, cnte = _get_hist()(idx0, idx1)
    (p1,) = _get_stage()(x, _pack_idx(idx0, idx1))
    edge_feat = _combine(p1, cnte)
    (p2,) = _get_stage()(edge_feat, _pack_idx(idx1, idx0))
    return _final(p2, cntv, W, b.reshape(1, D))


# speedup vs baseline: 10.5759x; 1.0385x over previous
import functools

import jax
import jax.numpy as jnp
from jax import lax
from jax.experimental import pallas as pl
from jax.experimental.pallas import tpu as pltpu
from jax.experimental.pallas import tpu_sc as plsc

N = 10000
D = 128
E = 320000
NC, NS, L = 2, 16, 16
NW = NC * NS
EPW = E // NW
CHUNK = 80
ITERS = EPW // CHUNK
RPT = 640
ZR = 40
NZ_FULL = RPT // ZR
NZ_LAST = (N - (NS - 1) * RPT) // ZR
CK = 128
NCK = 79
EPP = NCK * CK
NDUMP = N + 8


def _mesh():
    return plsc.VectorSubcoreMesh(core_axis_name="c", subcore_axis_name="s",
                                  num_cores=NC, num_subcores=NS)


@functools.lru_cache(maxsize=None)
def _get_hist():
    out_type = [jax.ShapeDtypeStruct((NC, N, D), jnp.float32),
                jax.ShapeDtypeStruct((NC, N, D), jnp.float32)]
    scratch = [
        pltpu.VMEM((2, CK), jnp.int32),
        pltpu.VMEM((CK, D), jnp.float32),
        pltpu.VMEM((ZR, D), jnp.float32),
        pltpu.VMEM_SHARED((NDUMP, D), jnp.float32),
    ]

    def body(pidx_h, cntv_h, cnte_h, idx_v, ones_v, zero_v, cw_sh):
        c = lax.axis_index("c")
        s = lax.axis_index("s")
        wid = c * NS + s
        row0 = s * RPT
        nz = jnp.where(s == NS - 1, NZ_LAST, NZ_FULL)

        def fill(i, _):
            for j in range(D // L):
                ones_v[i, pl.ds(j * L, L)] = jnp.ones((L,), jnp.float32)
            return _
        lax.fori_loop(0, CK, fill, 0)

        def fill_zero(i, _):
            for j in range(D // L):
                zero_v[i, pl.ds(j * L, L)] = jnp.zeros((L,), jnp.float32)
            return _
        lax.fori_loop(0, ZR, fill_zero, 0)

        chunk0 = wid * NCK

        def one_pass(slot, dst_h):
            def zcopy(i, _):
                pltpu.sync_copy(zero_v, cw_sh.at[pl.ds(row0 + i * ZR, ZR)])
                return _
            lax.fori_loop(0, nz, zcopy, 0)
            plsc.subcore_barrier()

            def step(i, _):
                pltpu.sync_copy(pidx_h.at[chunk0 + i], idx_v)
                pltpu.sync_copy(ones_v, cw_sh.at[idx_v.at[slot]], add=True)
                return _
            lax.fori_loop(0, NCK, step, 0)
            plsc.subcore_barrier()

            def out_copy(i, _):
                r = row0 + i * ZR
                pltpu.sync_copy(cw_sh.at[pl.ds(r, ZR)], dst_h.at[c, pl.ds(r, ZR)])
                return _
            lax.fori_loop(0, nz, out_copy, 0)
            plsc.subcore_barrier()

        one_pass(0, cntv_h)
        one_pass(1, cnte_h)

    return pl.kernel(body, out_type=out_type, mesh=_mesh(),
                     scratch_types=scratch)


@functools.lru_cache(maxsize=None)
def _get_stage():
    out_type = [jax.ShapeDtypeStruct((NC, N, D), jnp.float32)]
    scratch = [
        pltpu.VMEM((2, 2, CK), jnp.int32),
        pltpu.VMEM((2, CK, D), jnp.float32),
        pltpu.VMEM((ZR, D), jnp.float32),
        pltpu.VMEM_SHARED((NDUMP, D), jnp.float32),
        pltpu.SemaphoreType.DMA,
        pltpu.SemaphoreType.DMA,
    ]

    def body(table_h, pidx_h, part_h, idx_v, rows_v, zero_v, acc_sh,
             sem0, sem1):
        c = lax.axis_index("c")
        s = lax.axis_index("s")
        wid = c * NS + s
        row0 = s * RPT
        nz = jnp.where(s == NS - 1, NZ_LAST, NZ_FULL)
        sems = (sem0, sem1)

        def fill_zero(i, _):
            for j in range(D // L):
                zero_v[i, pl.ds(j * L, L)] = jnp.zeros((L,), jnp.float32)
            return _
        lax.fori_loop(0, ZR, fill_zero, 0)

        def zcopy(i, _):
            pltpu.sync_copy(zero_v, acc_sh.at[pl.ds(row0 + i * ZR, ZR)])
            return _
        lax.fori_loop(0, nz, zcopy, 0)
        plsc.subcore_barrier()

        chunk0 = wid * NCK

        def load_and_fire(i, buf):
            pltpu.sync_copy(pidx_h.at[chunk0 + i], idx_v.at[buf])
            return pltpu.async_copy(table_h.at[idx_v.at[buf, 0]],
                                    rows_v.at[buf], sems[buf])

        def drain(i, buf, cp):
            cp.wait()
            pltpu.sync_copy(rows_v.at[buf], acc_sh.at[idx_v.at[buf, 1]],
                            add=True)

        cp = load_and_fire(0, 0)
        for i in range(NCK):
            buf = i % 2
            nxt = load_and_fire(i + 1, 1 - buf) if i + 1 < NCK else None
            drain(i, buf, cp)
            cp = nxt
        plsc.subcore_barrier()

        def out_copy(i, _):
            r = row0 + i * ZR
            pltpu.sync_copy(acc_sh.at[pl.ds(r, ZR)], part_h.at[c, pl.ds(r, ZR)])
            return _
        lax.fori_loop(0, nz, out_copy, 0)

    return pl.kernel(body, out_type=out_type, mesh=_mesh(),
                     scratch_types=scratch)


def _pack_idx(g, s, pad_gather_zero=True):
    gw = g.reshape(NW, EPW)
    sw = s.reshape(NW, EPW)
    pad_g = (jnp.zeros((NW, EPP - EPW), jnp.int32) if pad_gather_zero
             else jnp.full((NW, EPP - EPW), N, jnp.int32))
    pad_s = jnp.full((NW, EPP - EPW), N, jnp.int32)
    gp = jnp.concatenate([gw, pad_g], axis=1).reshape(NW, NCK, 1, CK)
    sp = jnp.concatenate([sw, pad_s], axis=1).reshape(NW, NCK, 1, CK)
    return jnp.concatenate([gp, sp], axis=2).reshape(NW * NCK, 2, CK)


_BLK = 1000


def _recip(c_ref):
    cnt = c_ref[0, :, 0:1] + c_ref[1, :, 0:1]
    return jnp.where(cnt > 0, 1.0 / jnp.where(cnt > 0, cnt, 1.0), 0.0)


def _combine(parts, cnts):
    def body(p_ref, c_ref, o_ref):
        o_ref[...] = (p_ref[0] + p_ref[1]) * _recip(c_ref)

    return pl.pallas_call(
        body,
        grid=(N // _BLK,),
        in_specs=[pl.BlockSpec((NC, _BLK, D), lambda i: (0, i, 0)),
                  pl.BlockSpec((NC, _BLK, D), lambda i: (0, i, 0))],
        out_specs=pl.BlockSpec((_BLK, D), lambda i: (i, 0)),
        out_shape=jax.ShapeDtypeStruct((N, D), jnp.float32),
    )(parts, cnts)


def _final(parts, cnts, W, b2):
    def body(p_ref, c_ref, w_ref, b_ref, o_ref):
        sfeat = (p_ref[0] + p_ref[1]) * _recip(c_ref)
        o_ref[...] = jnp.dot(sfeat, w_ref[...],
                             preferred_element_type=jnp.float32) + b_ref[...]

    return pl.pallas_call(
        body,
        grid=(N // _BLK,),
        in_specs=[pl.BlockSpec((NC, _BLK, D), lambda i: (0, i, 0)),
                  pl.BlockSpec((NC, _BLK, D), lambda i: (0, i, 0)),
                  pl.BlockSpec((D, D), lambda i: (0, 0)),
                  pl.BlockSpec((1, D), lambda i: (0, 0))],
        out_specs=pl.BlockSpec((_BLK, D), lambda i: (i, 0)),
        out_shape=jax.ShapeDtypeStruct((N, D), jnp.float32),
    )(parts, cnts, W, b2)


def kernel(x, hyperedge_index, W, b):
    idx0 = hyperedge_index[0]
    idx1 = hyperedge_index[1]
    cntv, cnte = _get_hist()(_pack_idx(idx0, idx1, pad_gather_zero=False))
    (p1,) = _get_stage()(x, _pack_idx(idx0, idx1))
    edge_feat = _combine(p1, cnte)
    (p2,) = _get_stage()(edge_feat, _pack_idx(idx1, idx0))
    return _final(p2, cntv, W, b.reshape(1, D))
